# Initial kernel scaffold; baseline (speedup 1.0000x reference)
import jax
import jax.numpy as jnp
from jax.experimental import pallas as pl


def kernel(x, values):
    raise NotImplementedError("write your pallas kernel here")


# speedup vs baseline: 3.9527x; 3.9527x over previous
import jax
import jax.numpy as jnp
from jax.experimental import pallas as pl

_ROWS = 8192
_COLS = 8192
_BR = 256


def _diag_add_kernel(x_ref, v_ref, out_ref):
    g = pl.program_id(0)
    out_ref[...] = x_ref[...]
    vblock = v_ref[0, pl.ds(g * _BR, _BR)]
    rows = jax.lax.broadcasted_iota(jnp.int32, (_BR, _BR), 0)
    cols = jax.lax.broadcasted_iota(jnp.int32, (_BR, _BR), 1)
    diag = jnp.where(rows == cols, vblock[:, None], 0.0)
    out_ref[:, pl.ds(g * _BR, _BR)] = x_ref[:, pl.ds(g * _BR, _BR)] + diag


def kernel(x, values):
    v2d = values.reshape(1, _COLS)
    return pl.pallas_call(
        _diag_add_kernel,
        grid=(_ROWS // _BR,),
        in_specs=[
            pl.BlockSpec((_BR, _COLS), lambda g: (g, 0)),
            pl.BlockSpec((1, _COLS), lambda g: (0, 0)),
        ],
        out_specs=pl.BlockSpec((_BR, _COLS), lambda g: (g, 0)),
        out_shape=jax.ShapeDtypeStruct((_ROWS, _COLS), x.dtype),
    )(x, v2d)
